# baseline (speedup 1.0000x reference)
import jax
import jax.numpy as jnp
from jax import lax
from jax.experimental import pallas as pl
from jax.experimental.pallas import tpu as pltpu
from jax.experimental.pallas import tpu_sc as plsc

B, L, V = 64, 8192, 21
CUTOFF = 0.1
NEG_INF = float("-inf")
LANES = 16
VECS = L // LANES
ROWS_PER_W = 2
BLK = 8
WPOS = BLK * 128
NWIN = L // WPOS


def _tec_kernel(ent_hbm, loss_hbm, aa_hbm, wt_hbm, out_hbm,
                wt_v, loss_v, aa_v, eb0, eb1, out_v, lsem, wsem0, wsem1):
    nc = 2
    wid = lax.axis_index("s") * nc + lax.axis_index("c")
    lane = lax.iota(jnp.int32, LANES)
    b0 = wid * ROWS_PER_W

    pltpu.sync_copy(wt_hbm, wt_v)

    def _zero(j, _):
        out_v[j // 8, pl.ds((j % 8) * LANES, LANES)] = jnp.zeros(
            (LANES,), jnp.float32)
        return 0
    lax.fori_loop(0, VECS, _zero, 0)

    ninf = jnp.full((LANES,), NEG_INF, jnp.float32)
    zero_i = jnp.zeros((LANES,), jnp.int32)

    def _win_cp(bh, bl, w, buf, sem):
        return pltpu.make_async_copy(
            ent_hbm.at[:, bh, pl.ds(w * BLK, BLK), bl, :], buf, sem)

    def _insert(chain, s, iv):
        m1, m2, m3, i1, i2, i3 = chain
        g1 = s > m1
        n1 = jnp.where(g1, s, m1)
        d1 = jnp.where(g1, m1, s)
        j1 = jnp.where(g1, iv, i1)
        e1 = jnp.where(g1, i1, iv)
        g2 = d1 > m2
        n2 = jnp.where(g2, d1, m2)
        d2 = jnp.where(g2, m2, d1)
        j2 = jnp.where(g2, e1, i2)
        e2 = jnp.where(g2, i2, e1)
        g3 = d2 > m3
        n3 = jnp.where(g3, d2, m3)
        j3 = jnp.where(g3, e2, i3)
        return (n1, n2, n3, j1, j2, j3)

    def do_row(r, _):
        b = b0 + r
        bh = b // 8
        bl = b % 8
        cl = pltpu.make_async_copy(loss_hbm.at[b], loss_v, lsem)
        ca = pltpu.make_async_copy(aa_hbm.at[b], aa_v, lsem)
        cl.start()
        ca.start()
        _win_cp(bh, bl, 0, eb0, wsem0).start()
        cl.wait()
        ca.wait()

        def _proc(buf, w, carry):
            l0 = w * WPOS

            def _blk(c, inner):
                ca_, cb_ = inner
                cbase = l0 + c * 128
                cvec = zero_i + c
                for k in range(BLK):
                    o = k * LANES
                    olane = lane + o
                    wts = wt_v[pl.ds(cbase + o, LANES)]
                    g = plsc.load_gather(buf, [wts, cvec, olane])
                    s = loss_v[pl.ds(cbase + o, LANES)] - g
                    mut = aa_v[pl.ds(cbase + o, LANES)] != wts
                    s = jnp.where(mut, s, ninf)
                    iv = olane + cbase
                    if k % 2 == 0:
                        ca_ = _insert(ca_, s, iv)
                    else:
                        cb_ = _insert(cb_, s, iv)
                return ca_, cb_

            return lax.fori_loop(0, BLK, _blk, carry)

        def _pair(p, carry):
            w = 2 * p
            _win_cp(bh, bl, w, eb0, wsem0).wait()
            _win_cp(bh, bl, w + 1, eb1, wsem1).start()
            carry = _proc(eb0, w, carry)
            _win_cp(bh, bl, w + 1, eb1, wsem1).wait()

            @pl.when(p + 1 < NWIN // 2)
            def _():
                _win_cp(bh, bl, w + 2, eb0, wsem0).start()

            carry = _proc(eb1, w + 1, carry)
            return carry

        chain0 = (ninf, ninf, ninf, zero_i, zero_i, zero_i)
        ca_, cb_ = lax.fori_loop(0, NWIN // 2, _pair, (chain0, chain0))

        (a1, a2, a3, ai1, ai2, ai3) = ca_
        (q1, q2, q3, qi1, qi2, qi3) = cb_
        c1 = a1 > q3
        c2 = a2 > q2
        c3 = a3 > q1
        m1 = jnp.where(c1, a1, q3)
        m2 = jnp.where(c2, a2, q2)
        m3 = jnp.where(c3, a3, q1)
        i1 = jnp.where(c1, ai1, qi3)
        i2 = jnp.where(c2, ai2, qi2)
        i3 = jnp.where(c3, ai3, qi1)

        s1, j1 = plsc.sort_key_val(m1, i1)
        s2, j2 = plsc.sort_key_val(m2, i2)
        s3, j3 = plsc.sort_key_val(m3, i3)

        r2 = lax.rev(s2, (0,))
        rj2 = lax.rev(j2, (0,))
        c = s1 >= r2
        t = jnp.where(c, s1, r2)
        tj = jnp.where(c, j1, rj2)
        t, tj = plsc.sort_key_val(t, tj)

        r3 = lax.rev(s3, (0,))
        rj3 = lax.rev(j3, (0,))
        c = t >= r3
        u = jnp.where(c, t, r3)
        uj = jnp.where(c, tj, rj3)
        u, uj = plsc.sort_key_val(u, uj)

        keep = (lane >= LANES - 3) & (u > CUTOFF)
        sig = 1.0 / (1.0 + jnp.exp(-jnp.where(keep, u, 0.0)))

        uj_hi = uj >> 7
        uj_lo = uj & 127
        plsc.store_scatter(out_v, [uj_hi, uj_lo], sig, mask=keep)
        pltpu.sync_copy(out_v, out_hbm.at[bh, :, bl])
        plsc.store_scatter(out_v, [uj_hi, uj_lo],
                           jnp.zeros((LANES,), jnp.float32), mask=keep)
        return 0

    lax.fori_loop(0, ROWS_PER_W, do_row, 0)


@jax.jit
def _revor_sc(ent5, loss, aa_tensor, aa_wt):
    mesh = plsc.VectorSubcoreMesh(core_axis_name="c", subcore_axis_name="s")
    f = pl.kernel(
        _tec_kernel,
        mesh=mesh,
        out_type=jax.ShapeDtypeStruct((8, 64, 8, 128), jnp.float32),
        scratch_types=[
            pltpu.VMEM((L,), jnp.int32),
            pltpu.VMEM((L,), jnp.float32),
            pltpu.VMEM((L,), jnp.int32),
            pltpu.VMEM((V, BLK, 128), jnp.float32),
            pltpu.VMEM((V, BLK, 128), jnp.float32),
            pltpu.VMEM((64, 128), jnp.float32),
            pltpu.SemaphoreType.DMA,
            pltpu.SemaphoreType.DMA,
            pltpu.SemaphoreType.DMA,
        ],
        compiler_params=pltpu.CompilerParams(needs_layout_passes=False),
    )
    return f(ent5, loss, aa_tensor, aa_wt)


def kernel(entropy, loss, aa_tensor, aa_wt, max_step):
    del max_step
    ent5 = (entropy.reshape(8, 8, 64, 128, V)
            .transpose(4, 0, 2, 1, 3))
    out_nat = _revor_sc(ent5, loss, aa_tensor, aa_wt)
    return out_nat.transpose(0, 2, 1, 3).reshape(B, L)
